# baseline (speedup 1.0000x reference)
import jax
import jax.numpy as jnp
from jax import lax
from jax.experimental import pallas as pl
from jax.experimental.pallas import tpu as pltpu
from jax.experimental.pallas import tpu_sc as plsc

N_USERS = 20000
N_ITEMS = 30000
N_NODES = N_USERS + N_ITEMS
EMB = 64
N_EDGES = 800000

NUM_CORES = 2
NUM_SUBCORES = 16
ES = N_EDGES // NUM_SUBCORES
BLK = 2000
NBLK = ES // BLK
VPB = BLK // 16
CR = 6272
CPC = 4
NCHUNK = NUM_CORES * CPC
N_PAD = NCHUNK * CR
PT = CR // NUM_SUBCORES
GB = 64
CAP = 2048
NSB = CAP // GB

ROW_BLK = 2000


def _zero_accs(accs, zbuf, tb):
    for acc in accs:
        for q in range(PT // GB):
            pltpu.sync_copy(zbuf, acc.at[pl.ds(tb + q * GB, GB)])
        pltpu.sync_copy(zbuf.at[pl.ds(0, PT % GB)],
                        acc.at[pl.ds(tb + (PT // GB) * GB, PT % GB)])


def _drain_accs(accs, outs, tb, lo):
    for acc, out in zip(accs, outs):
        for q in range(PT // GB):
            pltpu.sync_copy(acc.at[pl.ds(tb + q * GB, GB)],
                            out.at[pl.ds(lo + tb + q * GB, GB)])
        pltpu.sync_copy(acc.at[pl.ds(tb + (PT // GB) * GB, PT % GB)],
                        out.at[pl.ds(lo + tb + (PT // GB) * GB, PT % GB)])


def _pipeline(nb, cidx, cval, crow2, xs, bufsets, semsets, semsc, accs):
    def fire(bb, si):
        idxsl = cidx.at[pl.ds(bb * GB, GB)]
        for i in range(3):
            pltpu.async_copy(xs[i].at[idxsl], bufsets[si][i], semsets[si][i])

    def scale_scatter(bb, si):
        base = bb * GB
        idxsl = cidx.at[pl.ds(base, GB)]
        rsl = crow2.at[bb]
        for i in range(3):
            xb = bufsets[si][i]
            pltpu.make_async_copy(xs[i].at[idxsl], xb, semsets[si][i]).wait()

            def grp(g, _, xb=xb):
                for j in range(4):
                    w = plsc.load_gather(
                        cval, [jnp.full((16,), base + g * 4 + j, jnp.int32)])
                    r = g * 4 + j
                    for k in range(4):
                        xb[r, pl.ds(k * 16, 16)] = xb[r, pl.ds(k * 16, 16)] * w
                return 0
            lax.fori_loop(0, GB // 4, grp, 0)
            pltpu.async_copy(xb, accs[i].at[rsl], semsc[si], add=True)

    def waitsc(si):
        rsl0 = crow2.at[0]
        for i in range(3):
            pltpu.make_async_copy(
                bufsets[si][i], accs[i].at[rsl0], semsc[si]).wait()

    @pl.when(nb > 0)
    def _():
        fire(0, 0)

    def pair(k, _):
        b0 = 2 * k
        b1 = b0 + 1

        @pl.when(b1 < nb)
        def _():
            @pl.when(k >= 1)
            def _():
                waitsc(1)
            fire(b1, 1)
        scale_scatter(b0, 0)

        @pl.when(b1 < nb)
        def _():
            @pl.when(b0 + 2 < nb)
            def _():
                waitsc(0)
                fire(b0 + 2, 0)
            scale_scatter(b1, 1)
        return 0
    lax.fori_loop(0, (nb + 1) // 2, pair, 0)

    @pl.when(nb >= 1)
    def _():
        waitsc(0)

    @pl.when(nb >= 2)
    def _():
        waitsc(1)


def _sc_l1_body(row_h, col_h, val_h, xv_h, xa_h, xt_h,
                ov_h, oa_h, ot_h, bidx_h, bval_h, brow_h, cnts_h,
                rowb, colb, valb, cidx, cval, crow2, cntv,
                xbv, xba, xbt, xcv, xca, xct, zbuf,
                seme, semb, semv, sema, semt, semv2, sema2, semt2,
                semsc0, semsc1, accv, acca, acct):
    c = lax.axis_index("c")
    s = lax.axis_index("s")

    zeros_f = jnp.zeros((16,), jnp.float32)
    zeros_i = jnp.zeros((16,), jnp.int32)

    def zb(j, _):
        for k in range(4):
            zbuf[j, pl.ds(k * 16, 16)] = zeros_f
        return 0
    lax.fori_loop(0, GB, zb, 0)

    accs = (accv, acca, acct)
    outs = (ov_h, oa_h, ot_h)
    xs = (xv_h, xa_h, xt_h)
    bufsets = ((xbv, xba, xbt), (xcv, xca, xct))
    semsets = ((semv, sema, semt), (semv2, sema2, semt2))
    semsc = (semsc0, semsc1)

    def chunk_body(ck, _):
        ckg = c * CPC + ck
        lo = ckg * CR
        tb = s * PT
        _zero_accs(accs, zbuf, tb)
        plsc.subcore_barrier()

        def blk_body(b, _):
            ebase = s * ES + b * BLK
            d1 = pltpu.async_copy(row_h.at[pl.ds(ebase, BLK)], rowb, seme)
            d2 = pltpu.async_copy(col_h.at[pl.ds(ebase, BLK)], colb, seme)
            d3 = pltpu.async_copy(val_h.at[pl.ds(ebase, BLK)], valb, seme)
            d1.wait()
            d2.wait()
            d3.wait()

            lane = lax.iota(jnp.int32, 16)

            def filt(v, cntc):
                rv = rowb[pl.ds(v * 16, 16)]
                m = (rv >= lo) & (rv < lo + CR)
                mi = m.astype(jnp.int32)
                incl = plsc.cumsum(mi)
                pos = jnp.where(m, cntc + incl - 1, jnp.int32(CAP) + lane)
                plsc.store_scatter(crow2, [pos // GB, pos % GB], rv - lo)
                plsc.store_scatter(cidx, [pos], colb[pl.ds(v * 16, 16)])
                plsc.store_scatter(cval, [pos], valb[pl.ds(v * 16, 16)])
                return cntc + jnp.sum(mi)
            cnt = lax.fori_loop(0, VPB, filt, jnp.int32(0))
            cnt_pad = ((cnt + GB - 1) // GB) * GB

            def padw(w, _):
                p = cnt + w * 16

                @pl.when(p < cnt_pad)
                def _():
                    crow2[p // GB, pl.ds(p % GB, 16)] = zeros_i
                    cidx[pl.ds(p, 16)] = zeros_i
                    cval[pl.ds(p, 16)] = zeros_f
                return 0
            lax.fori_loop(0, GB // 16, padw, 0)

            nb = cnt_pad // GB
            plsc.store_scatter(cntv, [jnp.full((16,), b, jnp.int32)],
                               jnp.full((16,), nb, jnp.int32),
                               mask=lane == 0)
            e1 = pltpu.async_copy(cidx.at[pl.ds(0, CAP)],
                                  bidx_h.at[ckg, s, b], semb)
            e2 = pltpu.async_copy(cval.at[pl.ds(0, CAP)],
                                  bval_h.at[ckg, s, b], semb)
            e3 = pltpu.async_copy(crow2.at[pl.ds(0, NSB)],
                                  brow_h.at[ckg, s, b], semb)

            _pipeline(nb, cidx, cval, crow2, xs, bufsets, semsets, semsc, accs)
            e1.wait()
            e2.wait()
            e3.wait()
            return 0
        lax.fori_loop(0, NBLK, blk_body, 0)
        pltpu.sync_copy(cntv, cnts_h.at[ckg, s])
        plsc.subcore_barrier()
        _drain_accs(accs, outs, tb, lo)
        plsc.subcore_barrier()
        return 0
    lax.fori_loop(0, CPC, chunk_body, 0)


def _sc_l2_body(bidx_h, bval_h, brow_h, cnts_h, xv_h, xa_h, xt_h,
                ov_h, oa_h, ot_h,
                cidx, cval, crow2, cntv,
                xbv, xba, xbt, xcv, xca, xct, zbuf,
                seme, semv, sema, semt, semv2, sema2, semt2,
                semsc0, semsc1, accv, acca, acct):
    c = lax.axis_index("c")
    s = lax.axis_index("s")

    zeros_f = jnp.zeros((16,), jnp.float32)

    def zb(j, _):
        for k in range(4):
            zbuf[j, pl.ds(k * 16, 16)] = zeros_f
        return 0
    lax.fori_loop(0, GB, zb, 0)

    accs = (accv, acca, acct)
    outs = (ov_h, oa_h, ot_h)
    xs = (xv_h, xa_h, xt_h)
    bufsets = ((xbv, xba, xbt), (xcv, xca, xct))
    semsets = ((semv, sema, semt), (semv2, sema2, semt2))
    semsc = (semsc0, semsc1)

    def chunk_body(ck, _):
        ckg = c * CPC + ck
        lo = ckg * CR
        tb = s * PT
        pltpu.sync_copy(cnts_h.at[ckg, s], cntv)
        _zero_accs(accs, zbuf, tb)
        plsc.subcore_barrier()

        def blk_body(b, _):
            lane = lax.iota(jnp.int32, 16)
            cl = cntv[pl.ds((b // 16) * 16, 16)]
            nb = jnp.sum(jnp.where(lane == b % 16, cl, jnp.int32(0)))
            d1 = pltpu.async_copy(bidx_h.at[ckg, s, b],
                                  cidx.at[pl.ds(0, CAP)], seme)
            d2 = pltpu.async_copy(bval_h.at[ckg, s, b],
                                  cval.at[pl.ds(0, CAP)], seme)
            d3 = pltpu.async_copy(brow_h.at[ckg, s, b],
                                  crow2.at[pl.ds(0, NSB)], seme)
            d1.wait()
            d2.wait()
            d3.wait()
            _pipeline(nb, cidx, cval, crow2, xs, bufsets, semsets, semsc, accs)
            return 0
        lax.fori_loop(0, NBLK, blk_body, 0)
        plsc.subcore_barrier()
        _drain_accs(accs, outs, tb, lo)
        plsc.subcore_barrier()
        return 0
    lax.fori_loop(0, CPC, chunk_body, 0)


_SC_PARAMS = pltpu.CompilerParams(
    needs_layout_passes=False, use_tc_tiling_on_sc=False)

_COMMON_SCRATCH = [
    pltpu.VMEM((CAP + 16,), jnp.int32),
    pltpu.VMEM((CAP + 16,), jnp.float32),
    pltpu.VMEM((NSB + 1, GB), jnp.int32),
    pltpu.VMEM((32,), jnp.int32),
    pltpu.VMEM((GB, EMB), jnp.float32),
    pltpu.VMEM((GB, EMB), jnp.float32),
    pltpu.VMEM((GB, EMB), jnp.float32),
    pltpu.VMEM((GB, EMB), jnp.float32),
    pltpu.VMEM((GB, EMB), jnp.float32),
    pltpu.VMEM((GB, EMB), jnp.float32),
    pltpu.VMEM((GB, EMB), jnp.float32),
]

_ACC_SCRATCH = [
    pltpu.VMEM_SHARED((CR, EMB), jnp.float32),
    pltpu.VMEM_SHARED((CR, EMB), jnp.float32),
    pltpu.VMEM_SHARED((CR, EMB), jnp.float32),
]


@jax.jit
def _sc_spmm_l1(row, col, val, xv, xa, xt):
    out_t = (
        [jax.ShapeDtypeStruct((N_PAD, EMB), jnp.float32)] * 3
        + [jax.ShapeDtypeStruct((NCHUNK, NUM_SUBCORES, NBLK, CAP), jnp.int32),
           jax.ShapeDtypeStruct((NCHUNK, NUM_SUBCORES, NBLK, CAP), jnp.float32),
           jax.ShapeDtypeStruct((NCHUNK, NUM_SUBCORES, NBLK, NSB, GB), jnp.int32),
           jax.ShapeDtypeStruct((NCHUNK, NUM_SUBCORES, 32), jnp.int32)]
    )
    mesh = plsc.VectorSubcoreMesh(core_axis_name="c", subcore_axis_name="s")
    f = pl.kernel(
        _sc_l1_body,
        out_type=out_t,
        mesh=mesh,
        compiler_params=_SC_PARAMS,
        scratch_types=(
            [pltpu.VMEM((BLK,), jnp.int32),
             pltpu.VMEM((BLK,), jnp.int32),
             pltpu.VMEM((BLK,), jnp.float32)]
            + _COMMON_SCRATCH
            + [pltpu.SemaphoreType.DMA] * 10
            + _ACC_SCRATCH
        ),
    )
    return f(row, col, val, xv, xa, xt)


@jax.jit
def _sc_spmm_l2(bidx, bval, brow, cnts, xv, xa, xt):
    out_t = [jax.ShapeDtypeStruct((N_PAD, EMB), jnp.float32)] * 3
    mesh = plsc.VectorSubcoreMesh(core_axis_name="c", subcore_axis_name="s")
    f = pl.kernel(
        _sc_l2_body,
        out_type=out_t,
        mesh=mesh,
        compiler_params=_SC_PARAMS,
        scratch_types=(
            _COMMON_SCRATCH
            + [pltpu.SemaphoreType.DMA] * 9
            + _ACC_SCRATCH
        ),
    )
    return f(bidx, bval, brow, cnts, xv, xa, xt)


def _dense3_body(sv, sa, st, wv, wa, wt, ov, oa, ot):
    for sref, wref, oref in ((sv, wv, ov), (sa, wa, oa), (st, wt, ot)):
        oref[...] = jax.nn.leaky_relu(
            lax.dot_general(sref[...], wref[...], (((1,), (1,)), ((), ())),
                            preferred_element_type=jnp.float32), 0.2)


def _dense3_fuse_body(sv, sa, st, wv, wa, wt, uid, out):
    acc = uid[...]
    for sref, wref in ((sv, wv), (sa, wa), (st, wt)):
        acc = acc + jax.nn.leaky_relu(
            lax.dot_general(sref[...], wref[...], (((1,), (1,)), ((), ())),
                            preferred_element_type=jnp.float32), 0.2)
    out[...] = acc


def _dense3(sv, sa, st, wv, wa, wt):
    grid = N_NODES // ROW_BLK
    row_spec = pl.BlockSpec((ROW_BLK, EMB), lambda i: (i, 0))
    w_spec = pl.BlockSpec((EMB, EMB), lambda i: (0, 0))
    return pl.pallas_call(
        _dense3_body,
        grid=(grid,),
        in_specs=[row_spec, row_spec, row_spec, w_spec, w_spec, w_spec],
        out_specs=[row_spec, row_spec, row_spec],
        out_shape=[jax.ShapeDtypeStruct((N_NODES, EMB), jnp.float32)] * 3,
    )(sv, sa, st, wv, wa, wt)


def _dense3_fuse(sv, sa, st, wv, wa, wt, uid):
    grid = N_NODES // ROW_BLK
    row_spec = pl.BlockSpec((ROW_BLK, EMB), lambda i: (i, 0))
    w_spec = pl.BlockSpec((EMB, EMB), lambda i: (0, 0))
    return pl.pallas_call(
        _dense3_fuse_body,
        grid=(grid,),
        in_specs=[row_spec, row_spec, row_spec, w_spec, w_spec, w_spec, row_spec],
        out_specs=row_spec,
        out_shape=jax.ShapeDtypeStruct((N_NODES, EMB), jnp.float32),
    )(sv, sa, st, wv, wa, wt, uid)


def kernel(adj_indices, adj_values, user_id_emb, item_id_emb,
           user_visual_emb, user_acoustic_emb, user_textual_emb,
           visual_feat, acoustic_feat, textual_feat, W_v, W_a, W_t):
    row = adj_indices[0]
    col = adj_indices[1]
    uid = jnp.concatenate([user_id_emb, item_id_emb], axis=0)
    vis = jnp.concatenate([user_visual_emb, visual_feat], axis=0)
    aco = jnp.concatenate([user_acoustic_emb, acoustic_feat], axis=0)
    tex = jnp.concatenate([user_textual_emb, textual_feat], axis=0)

    sv, sa, st, bidx, bval, brow, cnts = _sc_spmm_l1(
        row, col, adj_values, vis, aco, tex)
    vis, aco, tex = _dense3(sv, sa, st, W_v[0], W_a[0], W_t[0])
    sv, sa, st = _sc_spmm_l2(bidx, bval, brow, cnts, vis, aco, tex)
    fused = _dense3_fuse(sv, sa, st, W_v[1], W_a[1], W_t[1], uid)

    return (fused[:N_USERS], fused[N_USERS:])


# speedup vs baseline: 1.3252x; 1.3252x over previous
import functools

import jax
import jax.numpy as jnp
from jax import lax
from jax.experimental import pallas as pl
from jax.experimental.pallas import tpu as pltpu
from jax.experimental.pallas import tpu_sc as plsc

N_USERS = 20000
N_ITEMS = 30000
N_NODES = N_USERS + N_ITEMS
EMB = 64
N_EDGES = 800000

NUM_CORES = 2
NUM_SUBCORES = 16
ES = N_EDGES // NUM_SUBCORES
BLK = 2000
NBLK = ES // BLK
VPB = BLK // 16
CR = 6272
CPC = 4
N_PAD = NUM_CORES * CPC * CR
PT = CR // NUM_SUBCORES
GB = 64
CAP = 2048

ROW_BLK = 2000


def _sc_spmm_body(row_h, col_h, val_h, xv_h, xa_h, xt_h, ov_h, oa_h, ot_h,
                  rowb, colb, valb, cidx, cval, crow2,
                  xbv, xba, xbt, xcv, xca, xct, zbuf,
                  seme, semv, sema, semt, semv2, sema2, semt2,
                  semsc0, semsc1, accv, acca, acct):
    c = lax.axis_index("c")
    s = lax.axis_index("s")

    zeros_f = jnp.zeros((16,), jnp.float32)
    zeros_i = jnp.zeros((16,), jnp.int32)

    def zb(j, _):
        for k in range(4):
            zbuf[j, pl.ds(k * 16, 16)] = zeros_f
        return 0
    lax.fori_loop(0, GB, zb, 0)

    accs = (accv, acca, acct)
    outs = (ov_h, oa_h, ot_h)
    xs = (xv_h, xa_h, xt_h)
    xbs = (xbv, xba, xbt)

    def chunk_body(ck, _):
        lo = (c * CPC + ck) * CR
        tb = s * PT
        for acc in accs:
            sync = pltpu.sync_copy
            for q in range(PT // GB):
                sync(zbuf, acc.at[pl.ds(tb + q * GB, GB)])
            sync(zbuf.at[pl.ds(0, PT % GB)],
                 acc.at[pl.ds(tb + (PT // GB) * GB, PT % GB)])
        plsc.subcore_barrier()

        def blk_body(b, _):
            ebase = s * ES + b * BLK
            d1 = pltpu.async_copy(row_h.at[pl.ds(ebase, BLK)], rowb, seme)
            d2 = pltpu.async_copy(col_h.at[pl.ds(ebase, BLK)], colb, seme)
            d3 = pltpu.async_copy(val_h.at[pl.ds(ebase, BLK)], valb, seme)
            d1.wait()
            d2.wait()
            d3.wait()

            lane = lax.iota(jnp.int32, 16)

            def filt(v, cntv):
                rv = rowb[pl.ds(v * 16, 16)]
                m = (rv >= lo) & (rv < lo + CR)
                mi = m.astype(jnp.int32)
                incl = plsc.cumsum(mi)
                pos = jnp.where(m, cntv + incl - 1, jnp.int32(CAP) + lane)
                plsc.store_scatter(crow2, [pos // GB, pos % GB], rv - lo)
                plsc.store_scatter(cidx, [pos], colb[pl.ds(v * 16, 16)])
                plsc.store_scatter(cval, [pos], valb[pl.ds(v * 16, 16)])
                return cntv + plsc.all_reduce_population_count(m)
            cntv = lax.fori_loop(0, VPB, filt, jnp.zeros((16,), jnp.int32))
            cnt = jnp.max(cntv)
            cnt_pad = ((cnt + GB - 1) // GB) * GB

            def padw(w, _):
                p = cnt + w * 16

                @pl.when(p < cnt_pad)
                def _():
                    crow2[p // GB, pl.ds(p % GB, 16)] = zeros_i
                    cidx[pl.ds(p, 16)] = zeros_i
                    cval[pl.ds(p, 16)] = zeros_f
                return 0
            lax.fori_loop(0, GB // 16, padw, 0)

            nb = cnt_pad // GB

            bufsets = ((xbv, xba, xbt), (xcv, xca, xct))
            semsets = ((semv, sema, semt), (semv2, sema2, semt2))
            semsc = (semsc0, semsc1)

            def fire(bb, si):
                idxsl = cidx.at[pl.ds(bb * GB, GB)]
                for i in range(3):
                    pltpu.async_copy(xs[i].at[idxsl], bufsets[si][i], semsets[si][i])

            def scale_scatter(bb, si):
                base = bb * GB
                idxsl = cidx.at[pl.ds(base, GB)]
                rsl = crow2.at[bb]
                for i in range(3):
                    xb = bufsets[si][i]
                    pltpu.make_async_copy(
                        xs[i].at[idxsl], xb, semsets[si][i]).wait()

                    def grp(g, _, xb=xb):
                        for j in range(8):
                            w = plsc.load_gather(
                                cval, [jnp.full((16,), base + g * 8 + j, jnp.int32)])
                            r = g * 8 + j
                            for k in range(4):
                                xb[r, pl.ds(k * 16, 16)] = (
                                    xb[r, pl.ds(k * 16, 16)] * w)
                        return 0
                    lax.fori_loop(0, GB // 8, grp, 0)
                    pltpu.async_copy(xb, accs[i].at[rsl], semsc[si], add=True)

            def waitsc(si):
                rsl0 = crow2.at[0]
                for i in range(3):
                    pltpu.make_async_copy(
                        bufsets[si][i], accs[i].at[rsl0], semsc[si]).wait()

            @pl.when(nb > 0)
            def _():
                fire(0, 0)

            def pair(k, _):
                b0 = 2 * k
                b1 = b0 + 1

                @pl.when(b1 < nb)
                def _():
                    @pl.when(k >= 1)
                    def _():
                        waitsc(1)
                    fire(b1, 1)
                scale_scatter(b0, 0)

                @pl.when(b1 < nb)
                def _():
                    @pl.when(b0 + 2 < nb)
                    def _():
                        waitsc(0)
                        fire(b0 + 2, 0)
                    scale_scatter(b1, 1)
                return 0
            lax.fori_loop(0, (nb + 1) // 2, pair, 0)

            @pl.when(nb >= 1)
            def _():
                waitsc(0)

            @pl.when(nb >= 2)
            def _():
                waitsc(1)
            return 0
        lax.fori_loop(0, NBLK, blk_body, 0)
        plsc.subcore_barrier()

        for acc, out in zip(accs, outs):
            sync = pltpu.sync_copy
            for q in range(PT // GB):
                sync(acc.at[pl.ds(tb + q * GB, GB)], out.at[pl.ds(lo + tb + q * GB, GB)])
            sync(acc.at[pl.ds(tb + (PT // GB) * GB, PT % GB)],
                 out.at[pl.ds(lo + tb + (PT // GB) * GB, PT % GB)])
        plsc.subcore_barrier()
        return 0
    lax.fori_loop(0, CPC, chunk_body, 0)


@functools.partial(jax.jit, donate_argnums=())
def _sc_spmm(row, col, val, xv, xa, xt):
    out_t = [jax.ShapeDtypeStruct((N_PAD, EMB), jnp.float32)] * 3
    mesh = plsc.VectorSubcoreMesh(core_axis_name="c", subcore_axis_name="s")
    f = pl.kernel(
        _sc_spmm_body,
        out_type=out_t,
        mesh=mesh,
        compiler_params=pltpu.CompilerParams(needs_layout_passes=False, use_tc_tiling_on_sc=False),
        scratch_types=[
            pltpu.VMEM((BLK,), jnp.int32),
            pltpu.VMEM((BLK,), jnp.int32),
            pltpu.VMEM((BLK,), jnp.float32),
            pltpu.VMEM((CAP + 16,), jnp.int32),
            pltpu.VMEM((CAP + 16,), jnp.float32),
            pltpu.VMEM((CAP // GB + 1, GB), jnp.int32),
            pltpu.VMEM((GB, EMB), jnp.float32),
            pltpu.VMEM((GB, EMB), jnp.float32),
            pltpu.VMEM((GB, EMB), jnp.float32),
            pltpu.VMEM((GB, EMB), jnp.float32),
            pltpu.VMEM((GB, EMB), jnp.float32),
            pltpu.VMEM((GB, EMB), jnp.float32),
            pltpu.VMEM((GB, EMB), jnp.float32),
            pltpu.SemaphoreType.DMA,
            pltpu.SemaphoreType.DMA,
            pltpu.SemaphoreType.DMA,
            pltpu.SemaphoreType.DMA,
            pltpu.SemaphoreType.DMA,
            pltpu.SemaphoreType.DMA,
            pltpu.SemaphoreType.DMA,
            pltpu.SemaphoreType.DMA,
            pltpu.SemaphoreType.DMA,
            pltpu.VMEM_SHARED((CR, EMB), jnp.float32),
            pltpu.VMEM_SHARED((CR, EMB), jnp.float32),
            pltpu.VMEM_SHARED((CR, EMB), jnp.float32),
        ],
    )
    return f(row, col, val, xv, xa, xt)


def _dense3_body(sv, sa, st, wv, wa, wt, ov, oa, ot):
    for sref, wref, oref in ((sv, wv, ov), (sa, wa, oa), (st, wt, ot)):
        oref[...] = jax.nn.leaky_relu(
            lax.dot_general(sref[...], wref[...], (((1,), (1,)), ((), ())),
                            preferred_element_type=jnp.float32), 0.2)


def _dense3_fuse_body(sv, sa, st, wv, wa, wt, uid, out):
    acc = uid[...]
    for sref, wref in ((sv, wv), (sa, wa), (st, wt)):
        acc = acc + jax.nn.leaky_relu(
            lax.dot_general(sref[...], wref[...], (((1,), (1,)), ((), ())),
                            preferred_element_type=jnp.float32), 0.2)
    out[...] = acc


def _dense3(sv, sa, st, wv, wa, wt):
    grid = N_NODES // ROW_BLK
    row_spec = pl.BlockSpec((ROW_BLK, EMB), lambda i: (i, 0))
    w_spec = pl.BlockSpec((EMB, EMB), lambda i: (0, 0))
    return pl.pallas_call(
        _dense3_body,
        grid=(grid,),
        in_specs=[row_spec, row_spec, row_spec, w_spec, w_spec, w_spec],
        out_specs=[row_spec, row_spec, row_spec],
        out_shape=[jax.ShapeDtypeStruct((N_NODES, EMB), jnp.float32)] * 3,
    )(sv, sa, st, wv, wa, wt)


def _dense3_fuse(sv, sa, st, wv, wa, wt, uid):
    grid = N_NODES // ROW_BLK
    row_spec = pl.BlockSpec((ROW_BLK, EMB), lambda i: (i, 0))
    w_spec = pl.BlockSpec((EMB, EMB), lambda i: (0, 0))
    return pl.pallas_call(
        _dense3_fuse_body,
        grid=(grid,),
        in_specs=[row_spec, row_spec, row_spec, w_spec, w_spec, w_spec, row_spec],
        out_specs=row_spec,
        out_shape=jax.ShapeDtypeStruct((N_NODES, EMB), jnp.float32),
    )(sv, sa, st, wv, wa, wt, uid)


def kernel(adj_indices, adj_values, user_id_emb, item_id_emb,
           user_visual_emb, user_acoustic_emb, user_textual_emb,
           visual_feat, acoustic_feat, textual_feat, W_v, W_a, W_t):
    row = adj_indices[0]
    col = adj_indices[1]
    uid = jnp.concatenate([user_id_emb, item_id_emb], axis=0)
    vis = jnp.concatenate([user_visual_emb, visual_feat], axis=0)
    aco = jnp.concatenate([user_acoustic_emb, acoustic_feat], axis=0)
    tex = jnp.concatenate([user_textual_emb, textual_feat], axis=0)

    sv, sa, st = _sc_spmm(row, col, adj_values, vis, aco, tex)
    vis, aco, tex = _dense3(sv, sa, st, W_v[0], W_a[0], W_t[0])
    sv, sa, st = _sc_spmm(row, col, adj_values, vis, aco, tex)
    fused = _dense3_fuse(sv, sa, st, W_v[1], W_a[1], W_t[1], uid)

    return (fused[:N_USERS], fused[N_USERS:])
